# Initial kernel scaffold; baseline (speedup 1.0000x reference)
import jax
import jax.numpy as jnp
from jax.experimental import pallas as pl


def kernel(logits, pos_ratio):
    raise NotImplementedError("write your pallas kernel here")


# speedup vs baseline: 3.1826x; 3.1826x over previous
import jax
import jax.numpy as jnp
from jax.experimental import pallas as pl
from jax.experimental.pallas import tpu as pltpu

_N = 32768
_ROWS = 256
_COLS = 128
_HI_BITS = 0x3F7FFFFF


def _body(x_ref, k_ref, out_ref):
    x = x_ref[...]
    bits = jax.lax.bitcast_convert_type(x, jnp.int32)
    k = k_ref[0, 0]

    def step(_, lohi):
        lo, hi = lohi
        m = lo + (hi - lo + 1) // 2
        cnt = jnp.sum((bits >= m).astype(jnp.int32))
        ge = cnt >= k
        return jnp.where(ge, m, lo), jnp.where(ge, hi, m - 1)

    lo, _ = jax.lax.fori_loop(0, 30, step, (jnp.int32(0), jnp.int32(_HI_BITS)))
    t_bits = lo
    t = jax.lax.bitcast_convert_type(t_bits, jnp.float32)

    lx = jnp.log(x)
    l1x = jnp.log(1.0 - x)
    g = l1x - lx
    mask_gt = bits > t_bits
    s_neg = jnp.sum(-l1x)
    s_g_gt = jnp.sum(jnp.where(mask_gt, g, 0.0))
    c_gt = jnp.sum(mask_gt.astype(jnp.int32))
    g_t = jnp.log(1.0 - t) - jnp.log(t)
    total = s_neg + s_g_gt + (k - c_gt).astype(jnp.float32) * g_t
    out_ref[0, 0] = total / jnp.float32(_N)


def kernel(logits, pos_ratio):
    k = jnp.round(pos_ratio.reshape(()) * _N).astype(jnp.int32).reshape(1, 1)
    x = logits.reshape(_ROWS, _COLS)
    out = pl.pallas_call(
        _body,
        out_shape=jax.ShapeDtypeStruct((1, 1), jnp.float32),
        in_specs=[
            pl.BlockSpec(memory_space=pltpu.VMEM),
            pl.BlockSpec(memory_space=pltpu.SMEM),
        ],
        out_specs=pl.BlockSpec(memory_space=pltpu.SMEM),
    )(x, k)
    return out.reshape(())


# speedup vs baseline: 3.2291x; 1.0146x over previous
import jax
import jax.numpy as jnp
from jax.experimental import pallas as pl
from jax.experimental.pallas import tpu as pltpu

_N = 32768
_ROWS = 256
_COLS = 128
_HI_BITS = 0x3F7FFFFF


def _body(x_ref, k_ref, out_ref):
    x = x_ref[...]
    bits = jax.lax.bitcast_convert_type(x, jnp.int32)
    k = k_ref[0, 0]

    def step(_, lohi):
        lo, hi = lohi
        m = lo + (hi - lo + 1) // 2
        cnt = jnp.sum((bits >= m).astype(jnp.int32))
        ge = cnt >= k
        return jnp.where(ge, m, lo), jnp.where(ge, hi, m - 1)

    lo, _ = jax.lax.fori_loop(0, 30, step, (jnp.int32(0), jnp.int32(_HI_BITS)))
    t_bits = lo
    t = jax.lax.bitcast_convert_type(t_bits, jnp.float32)

    mask_gt = bits > t_bits
    y = jnp.where(mask_gt, x, 1.0 - x)
    s = jnp.sum(-jnp.log(y))
    c_gt = jnp.sum(mask_gt.astype(jnp.int32))
    g_t = jnp.log(1.0 - t) - jnp.log(t)
    total = s + (k - c_gt).astype(jnp.float32) * g_t
    out_ref[0, 0] = total / jnp.float32(_N)


def kernel(logits, pos_ratio):
    k = jnp.round(pos_ratio.reshape(()) * _N).astype(jnp.int32).reshape(1, 1)
    x = logits.reshape(_ROWS, _COLS)
    out = pl.pallas_call(
        _body,
        out_shape=jax.ShapeDtypeStruct((1, 1), jnp.float32),
        in_specs=[
            pl.BlockSpec(memory_space=pltpu.VMEM),
            pl.BlockSpec(memory_space=pltpu.SMEM),
        ],
        out_specs=pl.BlockSpec(memory_space=pltpu.SMEM),
    )(x, k)
    return out.reshape(())
